# baseline (speedup 1.0000x reference)
import jax
import jax.numpy as jnp
from jax import lax
from jax.experimental import pallas as pl
from jax.experimental.pallas import tpu as pltpu
from jax.experimental.pallas import tpu_sc as plsc

_D = 32
_H = 50
_B = 16384
_NC, _NS = 2, 16
_NW = _NC * _NS
_BT = 128
_TPW = _B // (_BT * _NW)
_KH = 2
_NGRP = _H // _KH


def _gather_body(table_hbm, idx_hbm, out_hbm, idx_v, idxT, grows, obuf, gsem, ssem):
    wid = lax.axis_index("s") * _NC + lax.axis_index("c")
    iota = lax.iota(jnp.int32, 16)
    rows16 = [iota + (16 * g) for g in range(8)]

    def bt_body(t, carry):
        btg = wid * _TPW + t
        b0 = btg * _BT

        pltpu.sync_copy(idx_hbm.at[pl.ds(b0, _BT)], idx_v)

        def hrow(h, c):
            col = jnp.full((16,), h, dtype=jnp.int32)
            for g in range(8):
                vec = plsc.load_gather(idx_v, [rows16[g], col])
                idxT[h, pl.ds(16 * g, 16)] = vec
            return c

        lax.fori_loop(0, _H, hrow, 0)

        def grp(g, c):
            gds = []
            for k in range(_KH):
                h = g * _KH + k
                gds.append(
                    pltpu.async_copy(
                        table_hbm.at[idxT.at[h]], grows.at[k], gsem.at[k]
                    )
                )
            for k in range(_KH):
                h = g * _KH + k
                gds[k].wait()
                for d in range(_D):
                    dcol = jnp.full((16,), d, dtype=jnp.int32)
                    for gg in range(8):
                        vec = plsc.load_gather(grows.at[k], [rows16[gg], dcol])
                        obuf[k, d, pl.ds(16 * gg, 16)] = vec
                for dt in range(4):
                    pltpu.async_copy(
                        obuf.at[k, pl.ds(dt * 8, 8)],
                        out_hbm.at[h, dt, btg],
                        ssem.at[k],
                    )
            for k in range(_KH):
                h = g * _KH + k
                for dt in range(4):
                    pltpu.make_async_copy(
                        obuf.at[k, pl.ds(dt * 8, 8)],
                        out_hbm.at[h, dt, btg],
                        ssem.at[k],
                    ).wait()
            return c

        lax.fori_loop(0, _NGRP, grp, 0)
        return carry

    lax.fori_loop(0, _TPW, bt_body, 0)


_gather = pl.kernel(
    _gather_body,
    out_type=jax.ShapeDtypeStruct((_H, 4, _B // _BT, 8, _BT), jnp.float32),
    mesh=plsc.VectorSubcoreMesh(core_axis_name="c", subcore_axis_name="s"),
    scratch_types=[
        pltpu.VMEM((_BT, _H), jnp.int32),
        pltpu.VMEM((_H, _BT), jnp.int32),
        pltpu.VMEM((_KH, _BT, _D), jnp.float32),
        pltpu.VMEM((_KH, _D, _BT), jnp.float32),
        pltpu.SemaphoreType.DMA((_KH,)),
        pltpu.SemaphoreType.DMA((_KH,)),
    ],
    compiler_params=pltpu.CompilerParams(
        use_tc_tiling_on_sc=False, needs_layout_passes=False
    ),
)


@jax.jit
def kernel(time, time_emb_weight):
    out5 = _gather(time_emb_weight, time)
    return out5.transpose(2, 4, 0, 1, 3).reshape(_B, _H, _D)


# speedup vs baseline: 1.7281x; 1.7281x over previous
import jax
import jax.numpy as jnp
from jax import lax
from jax.experimental import pallas as pl
from jax.experimental.pallas import tpu as pltpu
from jax.experimental.pallas import tpu_sc as plsc

_D = 32
_H = 50
_B = 16384
_NC, _NS = 2, 16
_NW = _NC * _NS
_BT = 128
_TPW = _B // (_BT * _NW)
_KH = 2
_NGRP = _H // _KH


def _gather_body(table_hbm, idx_hbm, out_hbm, idx_v, idxT, grows, obuf, gsem, ssem):
    wid = lax.axis_index("s") * _NC + lax.axis_index("c")
    iota = lax.iota(jnp.int32, 16)
    rows16 = [iota + (16 * g) for g in range(8)]

    def bt_body(t, carry):
        btg = wid * _TPW + t
        b0 = btg * _BT

        pltpu.sync_copy(idx_hbm.at[pl.ds(b0, _BT)], idx_v)

        @plsc.parallel_loop(0, _H, unroll=2)
        def _hrow(h):
            col = jnp.full((16,), h, dtype=jnp.int32)
            for g in range(8):
                vec = plsc.load_gather(idx_v, [rows16[g], col])
                idxT[h, pl.ds(16 * g, 16)] = vec

        def grp(g, c):
            gds = []
            for k in range(_KH):
                h = g * _KH + k
                gds.append(
                    pltpu.async_copy(
                        table_hbm.at[idxT.at[h]], grows.at[k], gsem.at[k]
                    )
                )
            for k in range(_KH):
                h = g * _KH + k
                gds[k].wait()

                @plsc.parallel_loop(0, _D, unroll=4)
                def _tr(d):
                    dcol = jnp.full((16,), d, dtype=jnp.int32)
                    for gg in range(8):
                        vec = plsc.load_gather(grows.at[k], [rows16[gg], dcol])
                        obuf[k, d, pl.ds(16 * gg, 16)] = vec
                for dt in range(4):
                    pltpu.async_copy(
                        obuf.at[k, pl.ds(dt * 8, 8)],
                        out_hbm.at[h, dt, btg],
                        ssem.at[k],
                    )
            for k in range(_KH):
                h = g * _KH + k
                for dt in range(4):
                    pltpu.make_async_copy(
                        obuf.at[k, pl.ds(dt * 8, 8)],
                        out_hbm.at[h, dt, btg],
                        ssem.at[k],
                    ).wait()
            return c

        lax.fori_loop(0, _NGRP, grp, 0)
        return carry

    lax.fori_loop(0, _TPW, bt_body, 0)


_gather = pl.kernel(
    _gather_body,
    out_type=jax.ShapeDtypeStruct((_H, 4, _B // _BT, 8, _BT), jnp.float32),
    mesh=plsc.VectorSubcoreMesh(core_axis_name="c", subcore_axis_name="s"),
    scratch_types=[
        pltpu.VMEM((_BT, _H), jnp.int32),
        pltpu.VMEM((_H, _BT), jnp.int32),
        pltpu.VMEM((_KH, _BT, _D), jnp.float32),
        pltpu.VMEM((_KH, _D, _BT), jnp.float32),
        pltpu.SemaphoreType.DMA((_KH,)),
        pltpu.SemaphoreType.DMA((_KH,)),
    ],
    compiler_params=pltpu.CompilerParams(
        use_tc_tiling_on_sc=False, needs_layout_passes=False
    ),
)


@jax.jit
def kernel(time, time_emb_weight):
    out5 = _gather(time_emb_weight, time)
    return out5.transpose(2, 4, 0, 1, 3).reshape(_B, _H, _D)
